# Initial kernel scaffold; baseline (speedup 1.0000x reference)
import jax
import jax.numpy as jnp
from jax.experimental import pallas as pl


def kernel(img, w0, b0, w1, b1, w2, b2):
    raise NotImplementedError("write your pallas kernel here")


# speedup vs baseline: 1.3313x; 1.3313x over previous
import functools

import jax
import jax.numpy as jnp
import numpy as np
from jax import lax
from jax.experimental import pallas as pl
from jax.experimental.pallas import tpu as pltpu


def _ru(x, m):
    return (x + m - 1) // m * m


def _body(x_ref, w0p_ref, w0l_ref, w12_ref, b_ref, s_ref, o_ref, acta, actb,
          *, H, W, K, p, pool, Ho, Wo, Cin, Cout, depth, Bblk, SEG, G, NT):
    HW = H * W
    OHW = Ho * Wo
    KK = K * K

    acta[...] = jnp.zeros_like(acta)
    actb[...] = jnp.zeros_like(actb)
    for b in range(Bblk):
        s0 = b * SEG + G
        acta[0:Cin, s0:s0 + HW] = x_ref[b]

    wc = lax.broadcasted_iota(jnp.int32, (1, NT), 1) % W
    kw_mask = {}
    for kw in range(K):
        off = kw - p
        if off != 0:
            kw_mask[kw] = jnp.logical_and(wc + off >= 0, wc + off < W)

    def slab(src, rows, b, n0, t):
        kh, kw = t // K, t % K
        d = (kh - p) * W + (kw - p)
        s0 = b * SEG + G + n0 + d
        v = src[0:rows, s0:s0 + NT]
        if kw != p:
            v = jnp.where(kw_mask[kw], v, jnp.zeros_like(v))
        return v

    ntiles = HW // NT

    for b in range(Bblk):
        for n in range(ntiles):
            n0 = n * NT
            acc = None
            for q in range(KK // 2):
                sl = jnp.concatenate(
                    [slab(acta, Cin, b, n0, 2 * q),
                     slab(acta, Cin, b, n0, 2 * q + 1)], axis=0)
                term = jnp.dot(w0p_ref[q], sl,
                               preferred_element_type=jnp.float32)
                acc = term if acc is None else acc + term
            if KK % 2:
                acc = acc + jnp.dot(w0l_ref[...],
                                    slab(acta, Cin, b, n0, KK - 1),
                                    preferred_element_type=jnp.float32)
            y = jnp.maximum(acc + b_ref[0], 0.0).astype(actb.dtype)
            actb[0:Cout, b * SEG + G + n0:b * SEG + G + n0 + NT] = y

    src, dst = actb, acta
    for l in range(1, depth):
        for b in range(Bblk):
            for n in range(ntiles):
                n0 = n * NT
                acc = None
                for t in range(KK):
                    term = jnp.dot(w12_ref[l - 1, t],
                                   slab(src, Cout, b, n0, t),
                                   preferred_element_type=jnp.float32)
                    acc = term if acc is None else acc + term
                y = jnp.maximum(acc + b_ref[l], 0.0).astype(src.dtype)
                dst[0:Cout, b * SEG + G + n0:b * SEG + G + n0 + NT] = y
        src, dst = dst, src

    for b in range(Bblk):
        base = b * SEG + G
        m = None
        for ph in range(pool):
            for pw in range(pool):
                d = ph * W + pw
                v = src[0:Cout, base + d:base + d + HW]
                m = v if m is None else jnp.maximum(m, v)
        pooled = jnp.dot(m, s_ref[...], preferred_element_type=jnp.float32)
        o_ref[:, b * OHW:(b + 1) * OHW] = pooled


def _pool_select(H, W, pool):
    Ho, Wo = H // pool, W // pool
    S = np.zeros((H * W, Ho * Wo), np.float32)
    for oh in range(Ho):
        for ow in range(Wo):
            S[(pool * oh) * W + pool * ow, oh * Wo + ow] = 1.0
    return jnp.asarray(S, jnp.bfloat16)


def _encoder(img, params, K, pool, batch_blocks, NT):
    B, Cin, H, W = img.shape
    Cout = params[0][0].shape[0]
    depth = len(params)
    p = K // 2
    Ho, Wo = H // pool, W // pool
    HW, OHW = H * W, Ho * Wo
    KK = K * K
    assert B % batch_blocks == 0 and HW % NT == 0 and NT % W == 0
    Bblk = B // batch_blocks
    guard = max(p, pool - 1) * (W + 1)
    G = _ru(guard, 128)
    SEG = G + _ru(HW + guard, 128)
    Cmax = max(Cin, Cout)

    x = img.reshape(B, Cin, HW).astype(jnp.bfloat16)
    w0 = params[0][0].astype(jnp.bfloat16)
    taps0 = [w0[:, :, t // K, t % K] for t in range(KK)]
    w0p = jnp.stack([jnp.concatenate([taps0[2 * q], taps0[2 * q + 1]], axis=1)
                     for q in range(KK // 2)])
    w0l = taps0[-1]
    w12 = jnp.stack([
        jnp.stack([params[l][0][:, :, t // K, t % K].astype(jnp.bfloat16)
                   for t in range(KK)]) for l in range(1, depth)])
    bias = jnp.stack([prm[1].astype(jnp.float32).reshape(Cout, 1)
                      for prm in params])
    sel = _pool_select(H, W, pool)

    out = pl.pallas_call(
        functools.partial(_body, H=H, W=W, K=K, p=p, pool=pool, Ho=Ho, Wo=Wo,
                          Cin=Cin, Cout=Cout, depth=depth, Bblk=Bblk, SEG=SEG,
                          G=G, NT=NT),
        out_shape=jax.ShapeDtypeStruct((batch_blocks * Cout, Bblk * OHW),
                                       jnp.float32),
        grid=(batch_blocks,),
        in_specs=[
            pl.BlockSpec((Bblk, Cin, HW), lambda i: (i, 0, 0)),
            pl.BlockSpec(w0p.shape, lambda i: (0, 0, 0)),
            pl.BlockSpec(w0l.shape, lambda i: (0, 0)),
            pl.BlockSpec(w12.shape, lambda i: (0, 0, 0, 0)),
            pl.BlockSpec(bias.shape, lambda i: (0, 0, 0)),
            pl.BlockSpec(sel.shape, lambda i: (0, 0)),
        ],
        out_specs=pl.BlockSpec((Cout, Bblk * OHW), lambda i: (i, 0)),
        scratch_shapes=[pltpu.VMEM((Cmax, Bblk * SEG), jnp.bfloat16),
                        pltpu.VMEM((Cmax, Bblk * SEG), jnp.bfloat16)],
        compiler_params=pltpu.CompilerParams(
            dimension_semantics=("parallel",)),
    )(x, w0p, w0l, w12, bias, sel)

    out = out.reshape(batch_blocks, Cout, Bblk, Ho, Wo)
    return jnp.transpose(out, (0, 2, 1, 3, 4)).reshape(B, Cout, Ho, Wo)


def kernel(img, w0, b0, w1, b1, w2, b2):
    params = [(w0, b0), (w1, b1), (w2, b2)]
    return _encoder(img, params, 3, 2, batch_blocks=32, NT=512)


# speedup vs baseline: 1.3524x; 1.0158x over previous
import functools

import jax
import jax.numpy as jnp
import numpy as np
from jax import lax
from jax.experimental import pallas as pl
from jax.experimental.pallas import tpu as pltpu


def _ru(x, m):
    return (x + m - 1) // m * m


def _body(x_ref, w0p_ref, w0l_ref, w12_ref, b_ref, s_ref, o_ref, acta, actb,
          *, H, W, K, p, pool, Ho, Wo, Cin, Cout, depth, Bblk, SEG, G, NT):
    HW = H * W
    OHW = Ho * Wo
    KK = K * K

    acta[...] = jnp.zeros_like(acta)
    actb[...] = jnp.zeros_like(actb)
    for b in range(Bblk):
        s0 = b * SEG + G
        acta[0:Cin, s0:s0 + HW] = x_ref[b]

    wc = lax.broadcasted_iota(jnp.int32, (1, NT), 1) % W
    kw_mask = {}
    for kw in range(K):
        off = kw - p
        if off != 0:
            kw_mask[kw] = jnp.logical_and(wc + off >= 0, wc + off < W)

    def slab(src, rows, b, n0, t):
        kh, kw = t // K, t % K
        d = (kh - p) * W + (kw - p)
        s0 = b * SEG + G + n0 + d
        v = src[0:rows, s0:s0 + NT]
        if kw != p:
            v = jnp.where(kw_mask[kw], v, jnp.zeros_like(v))
        return v

    ntiles = HW // NT

    for b in range(Bblk):
        for n in range(ntiles):
            n0 = n * NT
            acc = None
            for q in range(KK // 2):
                sl = jnp.concatenate(
                    [slab(acta, Cin, b, n0, 2 * q),
                     slab(acta, Cin, b, n0, 2 * q + 1)], axis=0)
                term = jnp.dot(w0p_ref[q], sl,
                               preferred_element_type=jnp.float32)
                acc = term if acc is None else acc + term
            if KK % 2:
                acc = acc + jnp.dot(w0l_ref[...],
                                    slab(acta, Cin, b, n0, KK - 1),
                                    preferred_element_type=jnp.float32)
            y = jnp.maximum(acc + b_ref[0], 0.0).astype(actb.dtype)
            actb[0:Cout, b * SEG + G + n0:b * SEG + G + n0 + NT] = y

    src, dst = actb, acta
    for l in range(1, depth):
        for b in range(Bblk):
            for n in range(ntiles):
                n0 = n * NT
                acc = None
                for t in range(KK):
                    term = jnp.dot(w12_ref[l - 1, t],
                                   slab(src, Cout, b, n0, t),
                                   preferred_element_type=jnp.float32)
                    acc = term if acc is None else acc + term
                y = jnp.maximum(acc + b_ref[l], 0.0).astype(src.dtype)
                dst[0:Cout, b * SEG + G + n0:b * SEG + G + n0 + NT] = y
        src, dst = dst, src

    for b in range(Bblk):
        base = b * SEG + G
        m = None
        for ph in range(pool):
            for pw in range(pool):
                d = ph * W + pw
                v = src[0:Cout, base + d:base + d + HW]
                m = v if m is None else jnp.maximum(m, v)
        pooled = jnp.dot(m, s_ref[...], preferred_element_type=jnp.float32)
        o_ref[:, b * OHW:(b + 1) * OHW] = pooled


def _pool_select(H, W, pool):
    Ho, Wo = H // pool, W // pool
    S = np.zeros((H * W, Ho * Wo), np.float32)
    for oh in range(Ho):
        for ow in range(Wo):
            S[(pool * oh) * W + pool * ow, oh * Wo + ow] = 1.0
    return jnp.asarray(S, jnp.bfloat16)


def _encoder(img, params, K, pool, batch_blocks, NT):
    B, Cin, H, W = img.shape
    Cout = params[0][0].shape[0]
    depth = len(params)
    p = K // 2
    Ho, Wo = H // pool, W // pool
    HW, OHW = H * W, Ho * Wo
    KK = K * K
    assert B % batch_blocks == 0 and HW % NT == 0 and NT % W == 0
    Bblk = B // batch_blocks
    guard = max(p, pool - 1) * (W + 1)
    G = _ru(guard, 128)
    SEG = G + _ru(HW + guard, 128)
    Cmax = max(Cin, Cout)

    x = img.reshape(B, Cin, HW).astype(jnp.bfloat16)
    w0 = params[0][0].astype(jnp.bfloat16)
    taps0 = [w0[:, :, t // K, t % K] for t in range(KK)]
    w0p = jnp.stack([jnp.concatenate([taps0[2 * q], taps0[2 * q + 1]], axis=1)
                     for q in range(KK // 2)])
    w0l = taps0[-1]
    w12 = jnp.stack([
        jnp.stack([params[l][0][:, :, t // K, t % K].astype(jnp.bfloat16)
                   for t in range(KK)]) for l in range(1, depth)])
    bias = jnp.stack([prm[1].astype(jnp.float32).reshape(Cout, 1)
                      for prm in params])
    sel = _pool_select(H, W, pool)

    out = pl.pallas_call(
        functools.partial(_body, H=H, W=W, K=K, p=p, pool=pool, Ho=Ho, Wo=Wo,
                          Cin=Cin, Cout=Cout, depth=depth, Bblk=Bblk, SEG=SEG,
                          G=G, NT=NT),
        out_shape=jax.ShapeDtypeStruct((batch_blocks * Cout, Bblk * OHW),
                                       jnp.float32),
        grid=(batch_blocks,),
        in_specs=[
            pl.BlockSpec((Bblk, Cin, HW), lambda i: (i, 0, 0)),
            pl.BlockSpec(w0p.shape, lambda i: (0, 0, 0)),
            pl.BlockSpec(w0l.shape, lambda i: (0, 0)),
            pl.BlockSpec(w12.shape, lambda i: (0, 0, 0, 0)),
            pl.BlockSpec(bias.shape, lambda i: (0, 0, 0)),
            pl.BlockSpec(sel.shape, lambda i: (0, 0)),
        ],
        out_specs=pl.BlockSpec((Cout, Bblk * OHW), lambda i: (i, 0)),
        scratch_shapes=[pltpu.VMEM((Cmax, Bblk * SEG), jnp.bfloat16),
                        pltpu.VMEM((Cmax, Bblk * SEG), jnp.bfloat16)],
        compiler_params=pltpu.CompilerParams(
            dimension_semantics=("parallel",)),
    )(x, w0p, w0l, w12, bias, sel)

    out = out.reshape(batch_blocks, Cout, Bblk, Ho, Wo)
    return jnp.transpose(out, (0, 2, 1, 3, 4)).reshape(B, Cout, Ho, Wo)


def kernel(img, w0, b0, w1, b1, w2, b2):
    params = [(w0, b0), (w1, b1), (w2, b2)]
    return _encoder(img, params, 3, 2, batch_blocks=16, NT=512)


# speedup vs baseline: 1.3632x; 1.0080x over previous
import functools

import jax
import jax.numpy as jnp
import numpy as np
from jax import lax
from jax.experimental import pallas as pl
from jax.experimental.pallas import tpu as pltpu


def _ru(x, m):
    return (x + m - 1) // m * m


def _body(x_ref, w0p_ref, w0l_ref, w12_ref, b_ref, p_ref, s_ref, o_ref,
          acta, actb, *, H, W, WP, K, p, pool, Ho, Wo, Cin, Cout, depth,
          Bblk, SEG, G, NT):
    HWP = H * WP
    OHW = Ho * Wo
    KK = K * K
    ntiles = HWP // NT

    acta[...] = jnp.zeros_like(acta)
    actb[...] = jnp.zeros_like(actb)
    for b in range(Bblk):
        base = b * SEG + G
        xb = x_ref[b].astype(jnp.bfloat16)
        xp = jnp.dot(xb, p_ref[...], preferred_element_type=jnp.float32)
        acta[0:Cin, base:base + HWP] = xp.astype(jnp.bfloat16)

    pad_mask = []
    for n in range(ntiles):
        wc = (lax.broadcasted_iota(jnp.int32, (1, NT), 1) + n * NT) % WP
        pad_mask.append(jnp.logical_and(wc >= 1, wc <= W))

    def slab(src, rows, b, n0, t):
        kh, kw = t // K, t % K
        d = (kh - p) * WP + (kw - p)
        s0 = b * SEG + G + n0 + d
        return src[0:rows, s0:s0 + NT]

    def finish(acc, l, n):
        y = jnp.maximum(acc + b_ref[l], 0.0)
        if l < depth - 1:
            y = jnp.where(pad_mask[n], y, 0.0)
        return y.astype(acta.dtype)

    for b in range(Bblk):
        for n in range(ntiles):
            n0 = n * NT
            acc = None
            for q in range(KK // 2):
                sl = jnp.concatenate(
                    [slab(acta, Cin, b, n0, 2 * q),
                     slab(acta, Cin, b, n0, 2 * q + 1)], axis=0)
                term = jnp.dot(w0p_ref[q], sl,
                               preferred_element_type=jnp.float32)
                acc = term if acc is None else acc + term
            if KK % 2:
                acc = acc + jnp.dot(w0l_ref[...],
                                    slab(acta, Cin, b, n0, KK - 1),
                                    preferred_element_type=jnp.float32)
            actb[0:Cout, b * SEG + G + n0:b * SEG + G + n0 + NT] = \
                finish(acc, 0, n)

    src, dst = actb, acta
    for l in range(1, depth):
        for b in range(Bblk):
            for n in range(ntiles):
                n0 = n * NT
                acc = None
                for t in range(KK):
                    term = jnp.dot(w12_ref[l - 1, t],
                                   slab(src, Cout, b, n0, t),
                                   preferred_element_type=jnp.float32)
                    acc = term if acc is None else acc + term
                dst[0:Cout, b * SEG + G + n0:b * SEG + G + n0 + NT] = \
                    finish(acc, l, n)
        src, dst = dst, src

    for b in range(Bblk):
        base = b * SEG + G
        m = None
        for ph in range(pool):
            for pw in range(pool):
                d = ph * WP + pw
                v = src[0:Cout, base + d:base + d + HWP]
                m = v if m is None else jnp.maximum(m, v)
        pooled = jnp.dot(m, s_ref[...], preferred_element_type=jnp.float32)
        o_ref[:, b * OHW:(b + 1) * OHW] = pooled


def _place_matrix(H, W, WP):
    P = np.zeros((H * W, H * WP), np.float32)
    for h in range(H):
        for w in range(W):
            P[h * W + w, h * WP + w + 1] = 1.0
    return jnp.asarray(P, jnp.bfloat16)


def _pool_select(H, W, WP, pool):
    Ho, Wo = H // pool, W // pool
    S = np.zeros((H * WP, Ho * Wo), np.float32)
    for oh in range(Ho):
        for ow in range(Wo):
            S[(pool * oh) * WP + pool * ow + 1, oh * Wo + ow] = 1.0
    return jnp.asarray(S, jnp.bfloat16)


def _encoder(img, params, K, pool, batch_blocks, NT):
    B, Cin, H, W = img.shape
    Cout = params[0][0].shape[0]
    depth = len(params)
    p = K // 2
    WP = W + 4
    Ho, Wo = H // pool, W // pool
    HW, HWP, OHW = H * W, H * WP, Ho * Wo
    KK = K * K
    assert B % batch_blocks == 0 and HWP % NT == 0
    Bblk = B // batch_blocks
    guard = max(p, pool - 1) * (WP + 1)
    G = _ru(guard, 128)
    SEG = G + _ru(HWP + guard, 128)
    Cmax = max(Cin, Cout)

    x = img.reshape(B, Cin, HW)
    w0 = params[0][0].astype(jnp.bfloat16)
    taps0 = [w0[:, :, t // K, t % K] for t in range(KK)]
    w0p = jnp.stack([jnp.concatenate([taps0[2 * q], taps0[2 * q + 1]], axis=1)
                     for q in range(KK // 2)])
    w0l = taps0[-1]
    w12 = jnp.stack([
        jnp.stack([params[l][0][:, :, t // K, t % K].astype(jnp.bfloat16)
                   for t in range(KK)]) for l in range(1, depth)])
    bias = jnp.stack([prm[1].astype(jnp.float32).reshape(Cout, 1)
                      for prm in params])
    place = _place_matrix(H, W, WP)
    sel = _pool_select(H, W, WP, pool)

    out = pl.pallas_call(
        functools.partial(_body, H=H, W=W, WP=WP, K=K, p=p, pool=pool, Ho=Ho,
                          Wo=Wo, Cin=Cin, Cout=Cout, depth=depth, Bblk=Bblk,
                          SEG=SEG, G=G, NT=NT),
        out_shape=jax.ShapeDtypeStruct((batch_blocks * Cout, Bblk * OHW),
                                       jnp.float32),
        grid=(batch_blocks,),
        in_specs=[
            pl.BlockSpec((Bblk, Cin, HW), lambda i: (i, 0, 0)),
            pl.BlockSpec(w0p.shape, lambda i: (0, 0, 0)),
            pl.BlockSpec(w0l.shape, lambda i: (0, 0)),
            pl.BlockSpec(w12.shape, lambda i: (0, 0, 0, 0)),
            pl.BlockSpec(bias.shape, lambda i: (0, 0, 0)),
            pl.BlockSpec(place.shape, lambda i: (0, 0)),
            pl.BlockSpec(sel.shape, lambda i: (0, 0)),
        ],
        out_specs=pl.BlockSpec((Cout, Bblk * OHW), lambda i: (i, 0)),
        scratch_shapes=[pltpu.VMEM((Cmax, Bblk * SEG), jnp.bfloat16),
                        pltpu.VMEM((Cmax, Bblk * SEG), jnp.bfloat16)],
        compiler_params=pltpu.CompilerParams(
            dimension_semantics=("parallel",)),
    )(x, w0p, w0l, w12, bias, place, sel)

    out = out.reshape(batch_blocks, Cout, Bblk, Ho, Wo)
    return jnp.transpose(out, (0, 2, 1, 3, 4)).reshape(B, Cout, Ho, Wo)


def kernel(img, w0, b0, w1, b1, w2, b2):
    params = [(w0, b0), (w1, b1), (w2, b2)]
    return _encoder(img, params, 3, 2, batch_blocks=16, NT=384)


# speedup vs baseline: 1.3742x; 1.0080x over previous
import functools

import jax
import jax.numpy as jnp
import numpy as np
from jax import lax
from jax.experimental import pallas as pl
from jax.experimental.pallas import tpu as pltpu


def _ru(x, m):
    return (x + m - 1) // m * m


def _body(x_ref, w0p_ref, w0l_ref, w12_ref, b_ref, p_ref, s_ref, o_ref,
          acta, actb, *, H, W, WP, K, p, pool, Ho, Wo, Cin, Cout, depth,
          Bblk, SEG, G, NT):
    HWP = H * WP
    OHW = Ho * Wo
    KK = K * K
    ntiles = HWP // NT

    acta[...] = jnp.zeros_like(acta)
    actb[...] = jnp.zeros_like(actb)
    for b in range(Bblk):
        base = b * SEG + G
        xb = x_ref[b].astype(jnp.bfloat16)
        xp = jnp.dot(xb, p_ref[...], preferred_element_type=jnp.float32)
        acta[0:Cin, base:base + HWP] = xp.astype(jnp.bfloat16)

    pad_mask = []
    for n in range(ntiles):
        wc = (lax.broadcasted_iota(jnp.int32, (1, NT), 1) + n * NT) % WP
        pad_mask.append(jnp.logical_and(wc >= 1, wc <= W))

    def slab(src, rows, b, n0, t):
        kh, kw = t // K, t % K
        d = (kh - p) * WP + (kw - p)
        s0 = b * SEG + G + n0 + d
        return src[0:rows, s0:s0 + NT]

    def finish(acc, l, n):
        y = jnp.maximum(acc + b_ref[l], 0.0)
        if l < depth - 1:
            y = jnp.where(pad_mask[n], y, 0.0)
        return y.astype(acta.dtype)

    for b in range(Bblk):
        for n in range(ntiles):
            n0 = n * NT
            acc = None
            for q in range(KK // 2):
                sl = jnp.concatenate(
                    [slab(acta, Cin, b, n0, 2 * q),
                     slab(acta, Cin, b, n0, 2 * q + 1)], axis=0)
                term = jnp.dot(w0p_ref[q], sl,
                               preferred_element_type=jnp.float32)
                acc = term if acc is None else acc + term
            if KK % 2:
                acc = acc + jnp.dot(w0l_ref[...],
                                    slab(acta, Cin, b, n0, KK - 1),
                                    preferred_element_type=jnp.float32)
            actb[0:Cout, b * SEG + G + n0:b * SEG + G + n0 + NT] = \
                finish(acc, 0, n)

    src, dst = actb, acta
    for l in range(1, depth):
        for b in range(Bblk):
            for n in range(ntiles):
                n0 = n * NT
                acc = None
                for t in range(KK):
                    term = jnp.dot(w12_ref[l - 1, t],
                                   slab(src, Cout, b, n0, t),
                                   preferred_element_type=jnp.float32)
                    acc = term if acc is None else acc + term
                dst[0:Cout, b * SEG + G + n0:b * SEG + G + n0 + NT] = \
                    finish(acc, l, n)
        src, dst = dst, src

    for b in range(Bblk):
        base = b * SEG + G
        m = None
        for ph in range(pool):
            for pw in range(pool):
                d = ph * WP + pw
                v = src[0:Cout, base + d:base + d + HWP]
                m = v if m is None else jnp.maximum(m, v)
        pooled = jnp.dot(m, s_ref[...], preferred_element_type=jnp.float32)
        o_ref[b * Cout:(b + 1) * Cout, :] = pooled


def _place_matrix(H, W, WP):
    P = np.zeros((H * W, H * WP), np.float32)
    for h in range(H):
        for w in range(W):
            P[h * W + w, h * WP + w + 1] = 1.0
    return jnp.asarray(P, jnp.bfloat16)


def _pool_select(H, W, WP, pool):
    Ho, Wo = H // pool, W // pool
    S = np.zeros((H * WP, Ho * Wo), np.float32)
    for oh in range(Ho):
        for ow in range(Wo):
            S[(pool * oh) * WP + pool * ow + 1, oh * Wo + ow] = 1.0
    return jnp.asarray(S, jnp.bfloat16)


def _encoder(img, params, K, pool, batch_blocks, NT):
    B, Cin, H, W = img.shape
    Cout = params[0][0].shape[0]
    depth = len(params)
    p = K // 2
    WP = W + 4
    Ho, Wo = H // pool, W // pool
    HW, HWP, OHW = H * W, H * WP, Ho * Wo
    KK = K * K
    assert B % batch_blocks == 0 and HWP % NT == 0
    Bblk = B // batch_blocks
    guard = max(p, pool - 1) * (WP + 1)
    G = _ru(guard, 128)
    SEG = G + _ru(HWP + guard, 128)
    Cmax = max(Cin, Cout)

    x = img.reshape(B, Cin, HW)
    t0 = params[0][0].astype(jnp.bfloat16).transpose(2, 3, 0, 1).reshape(
        KK, Cout, Cin)
    w0p = t0[:KK - 1].reshape((KK - 1) // 2, 2, Cout, Cin).transpose(
        0, 2, 1, 3).reshape((KK - 1) // 2, Cout, 2 * Cin)
    w0l = t0[KK - 1]
    w12 = jnp.stack([
        params[l][0].astype(jnp.bfloat16).transpose(2, 3, 0, 1).reshape(
            KK, Cout, Cout) for l in range(1, depth)])
    bias = jnp.stack([prm[1].astype(jnp.float32).reshape(Cout, 1)
                      for prm in params])
    place = _place_matrix(H, W, WP)
    sel = _pool_select(H, W, WP, pool)

    out = pl.pallas_call(
        functools.partial(_body, H=H, W=W, WP=WP, K=K, p=p, pool=pool, Ho=Ho,
                          Wo=Wo, Cin=Cin, Cout=Cout, depth=depth, Bblk=Bblk,
                          SEG=SEG, G=G, NT=NT),
        out_shape=jax.ShapeDtypeStruct((B * Cout, OHW), jnp.float32),
        grid=(batch_blocks,),
        in_specs=[
            pl.BlockSpec((Bblk, Cin, HW), lambda i: (i, 0, 0)),
            pl.BlockSpec(w0p.shape, lambda i: (0, 0, 0)),
            pl.BlockSpec(w0l.shape, lambda i: (0, 0)),
            pl.BlockSpec(w12.shape, lambda i: (0, 0, 0, 0)),
            pl.BlockSpec(bias.shape, lambda i: (0, 0, 0)),
            pl.BlockSpec(place.shape, lambda i: (0, 0)),
            pl.BlockSpec(sel.shape, lambda i: (0, 0)),
        ],
        out_specs=pl.BlockSpec((Bblk * Cout, OHW), lambda i: (i, 0)),
        scratch_shapes=[pltpu.VMEM((Cmax, Bblk * SEG), jnp.bfloat16),
                        pltpu.VMEM((Cmax, Bblk * SEG), jnp.bfloat16)],
        compiler_params=pltpu.CompilerParams(
            dimension_semantics=("parallel",)),
    )(x, w0p, w0l, w12, bias, place, sel)

    return out.reshape(B, Cout, Ho, Wo)


def kernel(img, w0, b0, w1, b1, w2, b2):
    params = [(w0, b0), (w1, b1), (w2, b2)]
    return _encoder(img, params, 3, 2, batch_blocks=16, NT=384)
